# baseline (speedup 1.0000x reference)
import functools

import jax
import jax.numpy as jnp
from jax import lax
from jax.experimental import pallas as pl
from jax.experimental.pallas import tpu as pltpu
from jax.experimental.pallas import tpu_sc as plsc

_T = 100000
_J = 29
_NB = 30
_B = 4096
_OBS = 480
_H1, _H2, _H3 = 512, 256, 128
_ACT = 29

_NC, _NS, _L = 2, 16, 16
_NW = _NC * _NS
_BPW = _B // _NW

_WIDTHS = (_J, _J, _NB * 3, _NB * 4, _NB * 3, _NB * 3)


def _gather_body(ts_hbm, tab, out, idx_v, rv, sem):
    wid = lax.axis_index("s") * _NC + lax.axis_index("c")
    base = wid * _BPW
    pltpu.sync_copy(ts_hbm.at[pl.ds(base, _BPW)], idx_v)
    iota = lax.iota(jnp.int32, _L)

    def group(g, _):
        vec = jnp.minimum(idx_v[pl.ds(g * _L, _L)], _T - 1)
        for i in range(_L):
            t = lax.reduce_max(jnp.where(iota == i, vec, 0), (0,))
            r = g * _L + i
            pltpu.async_copy(tab.at[pl.ds(t, 1)], rv.at[pl.ds(r, 1)], sem)
        return ()

    lax.fori_loop(0, _BPW // _L, group, (), unroll=False)
    pltpu.make_async_copy(tab.at[pl.ds(0, _BPW)], rv, sem).wait()
    pltpu.sync_copy(rv, out.at[pl.ds(base, _BPW)])


def _make_gather(w):
    return pl.kernel(
        _gather_body,
        out_type=jax.ShapeDtypeStruct((_B, w), jnp.float32),
        mesh=plsc.VectorSubcoreMesh(core_axis_name="c", subcore_axis_name="s"),
        scratch_types=[pltpu.VMEM((_BPW,), jnp.int32),
                       pltpu.VMEM((_BPW, w), jnp.float32),
                       pltpu.SemaphoreType.DMA],
        compiler_params=pltpu.CompilerParams(needs_layout_passes=False),
    )


_gathers = tuple(_make_gather(w) for w in _WIDTHS)


def _elu(x):
    return jnp.where(x > 0, x, jnp.exp(jnp.minimum(x, 0.0)) - 1.0)


def _mlp_body(obs_ref, w1, b1, w2, b2, w3, b3, w4, b4, out_ref):
    h = _elu(jnp.dot(obs_ref[...], w1[...], preferred_element_type=jnp.float32)
             + b1[...])
    h = _elu(jnp.dot(h, w2[...], preferred_element_type=jnp.float32) + b2[...])
    h = _elu(jnp.dot(h, w3[...], preferred_element_type=jnp.float32) + b3[...])
    out_ref[...] = (jnp.dot(h, w4[...], preferred_element_type=jnp.float32)
                    + b4[...])


_BM = 512


def _mlp(obs, W1, b1, W2, b2, W3, b3, W4, b4):
    full = lambda r, c: pl.BlockSpec((r, c), lambda i: (0, 0))
    return pl.pallas_call(
        _mlp_body,
        grid=(_B // _BM,),
        in_specs=[
            pl.BlockSpec((_BM, _OBS), lambda i: (i, 0)),
            full(_OBS, _H1), full(1, _H1),
            full(_H1, _H2), full(1, _H2),
            full(_H2, _H3), full(1, _H3),
            full(_H3, _ACT), full(1, _ACT),
        ],
        out_specs=pl.BlockSpec((_BM, _ACT), lambda i: (i, 0)),
        out_shape=jax.ShapeDtypeStruct((_B, _ACT), jnp.float32),
    )(obs, W1, b1.reshape(1, _H1), W2, b2.reshape(1, _H2),
      W3, b3.reshape(1, _H3), W4, b4.reshape(1, _ACT))


def kernel(obs, W1, b1, W2, b2, W3, b3, W4, b4, joint_pos, joint_vel,
           body_pos_w, body_quat_w, body_lin_vel_w, body_ang_vel_w, time_step):
    ts = time_step.reshape(_B).astype(jnp.int32)
    tables = (joint_pos, joint_vel,
              body_pos_w.reshape(_T, _NB * 3),
              body_quat_w.reshape(_T, _NB * 4),
              body_lin_vel_w.reshape(_T, _NB * 3),
              body_ang_vel_w.reshape(_T, _NB * 3))
    g = tuple(gk(ts, tab) for gk, tab in zip(_gathers, tables))
    policy_out = _mlp(obs, W1, b1, W2, b2, W3, b3, W4, b4)
    return (policy_out,
            g[0], g[1],
            g[2].reshape(_B, _NB, 3),
            g[3].reshape(_B, _NB, 4),
            g[4].reshape(_B, _NB, 3),
            g[5].reshape(_B, _NB, 3))


# speedup vs baseline: 1.0061x; 1.0061x over previous
import functools

import jax
import jax.numpy as jnp
from jax import lax
from jax.experimental import pallas as pl
from jax.experimental.pallas import tpu as pltpu
from jax.experimental.pallas import tpu_sc as plsc

_T = 100000
_J = 29
_NB = 30
_B = 4096
_OBS = 480
_H1, _H2, _H3 = 512, 256, 128
_ACT = 29

_NC, _NS, _L = 2, 16, 16
_NW = _NC * _NS
_BPW = _B // _NW

_WIDTHS = (_J, _J, _NB * 3, _NB * 4, _NB * 3, _NB * 3)


def _gather_body(ts_hbm, t0, t1, t2, t3, t4, t5,
                 o0, o1, o2, o3, o4, o5,
                 idx_v, r0, r1, r2, r3, r4, r5, sem):
    wid = lax.axis_index("s") * _NC + lax.axis_index("c")
    base = wid * _BPW
    tabs = (t0, t1, t2, t3, t4, t5)
    outs = (o0, o1, o2, o3, o4, o5)
    rows = (r0, r1, r2, r3, r4, r5)
    pltpu.sync_copy(ts_hbm.at[pl.ds(base, _BPW)], idx_v)
    iota = lax.iota(jnp.int32, _L)

    def group(g, _):
        vec = jnp.minimum(idx_v[pl.ds(g * _L, _L)], _T - 1)
        for i in range(_L):
            t = lax.reduce_max(jnp.where(iota == i, vec, 0), (0,))
            r = g * _L + i
            for tab, rv in zip(tabs, rows):
                pltpu.async_copy(tab.at[pl.ds(t, 1)], rv.at[pl.ds(r, 1)],
                                 sem)
        return ()

    lax.fori_loop(0, _BPW // _L, group, (), unroll=False)
    for tab, rv in zip(tabs, rows):
        pltpu.make_async_copy(tab.at[pl.ds(0, _BPW)], rv, sem).wait()
    for rv, o in zip(rows, outs):
        pltpu.sync_copy(rv, o.at[pl.ds(base, _BPW)])


_gather = pl.kernel(
    _gather_body,
    out_type=tuple(jax.ShapeDtypeStruct((_B, w), jnp.float32)
                   for w in _WIDTHS),
    mesh=plsc.VectorSubcoreMesh(core_axis_name="c", subcore_axis_name="s"),
    scratch_types=[pltpu.VMEM((_BPW,), jnp.int32)]
    + [pltpu.VMEM((_BPW, w), jnp.float32) for w in _WIDTHS]
    + [pltpu.SemaphoreType.DMA],
    compiler_params=pltpu.CompilerParams(needs_layout_passes=False),
)


def _elu(x):
    return jnp.where(x > 0, x, jnp.exp(jnp.minimum(x, 0.0)) - 1.0)


def _mlp_body(obs_ref, w1, b1, w2, b2, w3, b3, w4, b4, out_ref):
    h = _elu(jnp.dot(obs_ref[...], w1[...], preferred_element_type=jnp.float32)
             + b1[...])
    h = _elu(jnp.dot(h, w2[...], preferred_element_type=jnp.float32) + b2[...])
    h = _elu(jnp.dot(h, w3[...], preferred_element_type=jnp.float32) + b3[...])
    out_ref[...] = (jnp.dot(h, w4[...], preferred_element_type=jnp.float32)
                    + b4[...])


_BM = 512


def _mlp(obs, W1, b1, W2, b2, W3, b3, W4, b4):
    full = lambda r, c: pl.BlockSpec((r, c), lambda i: (0, 0))
    return pl.pallas_call(
        _mlp_body,
        grid=(_B // _BM,),
        in_specs=[
            pl.BlockSpec((_BM, _OBS), lambda i: (i, 0)),
            full(_OBS, _H1), full(1, _H1),
            full(_H1, _H2), full(1, _H2),
            full(_H2, _H3), full(1, _H3),
            full(_H3, _ACT), full(1, _ACT),
        ],
        out_specs=pl.BlockSpec((_BM, _ACT), lambda i: (i, 0)),
        out_shape=jax.ShapeDtypeStruct((_B, _ACT), jnp.float32),
    )(obs, W1, b1.reshape(1, _H1), W2, b2.reshape(1, _H2),
      W3, b3.reshape(1, _H3), W4, b4.reshape(1, _ACT))


def kernel(obs, W1, b1, W2, b2, W3, b3, W4, b4, joint_pos, joint_vel,
           body_pos_w, body_quat_w, body_lin_vel_w, body_ang_vel_w, time_step):
    ts = time_step.reshape(_B).astype(jnp.int32)
    g = _gather(ts, joint_pos, joint_vel,
                body_pos_w.reshape(_T, _NB * 3),
                body_quat_w.reshape(_T, _NB * 4),
                body_lin_vel_w.reshape(_T, _NB * 3),
                body_ang_vel_w.reshape(_T, _NB * 3))
    policy_out = _mlp(obs, W1, b1, W2, b2, W3, b3, W4, b4)
    return (policy_out,
            g[0], g[1],
            g[2].reshape(_B, _NB, 3),
            g[3].reshape(_B, _NB, 4),
            g[4].reshape(_B, _NB, 3),
            g[5].reshape(_B, _NB, 3))
